# baseline (speedup 1.0000x reference)
import jax
import jax.numpy as jnp
from jax import lax
from jax.experimental import pallas as pl
from jax.experimental.pallas import tpu as pltpu
from jax.experimental.pallas import tpu_sc as plsc

N = 3072
R = 1024
NSEG = 8
NC, NS = 2, 16
NW = NC * NS
Q = 8
QSH = 3
C = 512
CSH = 9
L = 16
NPAD = N + 2 * L
BIG = 1 << 30


def _sqrt16(sq):
    sqs = jnp.maximum(sq, 1e-30)
    ii = plsc.bitcast(sqs, jnp.int32)
    ii = jnp.int32(0x5F3759DF) - (ii >> 1)
    y = plsc.bitcast(ii, jnp.float32)
    h = sqs * 0.5
    y = y * (1.5 - h * y * y)
    y = y * (1.5 - h * y * y)
    return sq * y


def _body(inp_hbm, tgt_hbm, msk_hbm, si_hbm,
          oe_hbm, op_hbm,
          tgt_v, msk_v, si_v, alive_v, segc_v, txc_v, tyc_v, tzc_v,
          bnd_v, bndc_v, qbuf, oe_v, op_v, sem0, sem1):
    cid = lax.axis_index("c")
    sid = lax.axis_index("s")
    wid = sid * NC + cid

    pltpu.sync_copy(tgt_hbm, tgt_v)
    pltpu.sync_copy(msk_hbm, msk_v)
    pltpu.sync_copy(si_hbm, si_v)

    iota = lax.iota(jnp.int32, L)
    lane0 = iota == 0
    zero = jnp.zeros((L,), jnp.float32)
    zeroi = jnp.zeros((L,), jnp.int32)

    def prefill_body(v, _=None):
        off = v * L
        alive_v[pl.ds(off, L)] = jnp.full((L,), N - 1, jnp.int32)
        segc_v[pl.ds(off, L)] = jnp.full((L,), -1, jnp.int32)
        txc_v[pl.ds(off, L)] = zero
        tyc_v[pl.ds(off, L)] = zero
        tzc_v[pl.ds(off, L)] = zero

    plsc.parallel_loop(0, NPAD // L, unroll=2)(prefill_body)

    bnd_v[...] = jnp.full((L,), N, jnp.int32)
    seg0 = plsc.load_gather(si_v, [zeroi])
    plsc.store_scatter(bnd_v, [seg0], zeroi, mask=lane0)

    def prep_body(v, base):
        j = v * L + iota
        ridx = j // 3
        rem = j - ridx * 3
        nxtr = jnp.minimum(ridx + jnp.where(rem == 2, 1, 0), R - 1)
        cur = plsc.load_gather(si_v, [ridx])
        nxt = plsc.load_gather(si_v, [nxtr])
        plsc.store_scatter(bnd_v, [nxt], j + 1, mask=nxt != cur)
        m16 = msk_v[pl.ds(v * L, L)]
        am = m16 > 0
        pc = plsc.all_reduce_population_count(am)
        cs = plsc.cumsum(m16)
        cidx = base + cs - m16
        plsc.store_scatter(alive_v, [cidx], j, mask=am)
        plsc.store_scatter(segc_v, [cidx], cur, mask=am)
        j3 = j * 3
        plsc.store_scatter(txc_v, [cidx], plsc.load_gather(tgt_v, [j3]),
                           mask=am)
        plsc.store_scatter(tyc_v, [cidx], plsc.load_gather(tgt_v, [j3 + 1]),
                           mask=am)
        plsc.store_scatter(tzc_v, [cidx], plsc.load_gather(tgt_v, [j3 + 2]),
                           mask=am)
        return base + pc

    base_vec = plsc.parallel_loop(0, N // L, unroll=4, carry=zeroi)(prep_body)
    na = jnp.max(base_vec)

    bnd_v[...] = lax.rev(-plsc.cummax(-lax.rev(bnd_v[...], (0,))), (0,))

    bndc_v[...] = jnp.full((L,), BIG, jnp.int32)
    seg0c = plsc.load_gather(segc_v, [zeroi])
    plsc.store_scatter(bndc_v, [seg0c], zeroi, mask=lane0 & (seg0c >= 0))
    segl = plsc.load_gather(segc_v, [jnp.maximum(na - 1, 0) + zeroi])
    plsc.store_scatter(bndc_v, [segl + 1], na + zeroi,
                       mask=lane0 & (segl >= 0))

    def bndc_body(v, _=None):
        j = v * L + iota
        cur = segc_v[pl.ds(v * L, L)]
        nxt = segc_v[pl.ds(v * L + 1, L)]
        plsc.store_scatter(bndc_v, [nxt], j + 1,
                           mask=(nxt != cur) & (nxt >= 0))

    plsc.parallel_loop(0, (na + L) >> 4, unroll=2)(bndc_body)
    bndc_v[...] = lax.rev(-plsc.cummax(-lax.rev(bndc_v[...], (0,))), (0,))

    nq = (na + (Q - 1)) >> QSH
    myq = jnp.maximum((nq - wid + NW - 1) // NW, 0)

    def rowid(p):
        return jnp.max(plsc.load_gather(alive_v, [p + zeroi]))

    def chunk_bounds(q):
        b4 = q * Q
        sf = plsc.load_gather(segc_v, [b4 + zeroi])
        sl = plsc.load_gather(segc_v,
                              [jnp.minimum(b4 + (Q - 1), na - 1) + zeroi])
        s_v = plsc.load_gather(bnd_v, [sf])
        e_v = plsc.load_gather(bnd_v, [sl + 1])
        return jnp.max(s_v >> CSH), jnp.max((e_v + (C - 1)) >> CSH), sf, sl

    def issue(q, buf_off, sem):
        c0, c1, _, _ = chunk_bounds(q)
        b4 = q * Q
        rids = [rowid(b4 + r) for r in range(Q)]

        def issue_c(c, _):
            for r in range(Q):
                pltpu.async_copy(
                    inp_hbm.at[rids[r], pl.ds(c * C, C)],
                    qbuf.at[pl.ds(buf_off + r * N + c * C, C)], sem)
            return 0

        lax.fori_loop(c0, c1, issue_c, 0)
        return c0, c1

    def drain(c0, c1, sem):
        def drain_c(c, _):
            for r in range(Q):
                pltpu.make_async_copy(inp_hbm.at[0, pl.ds(0, C)],
                                      qbuf.at[pl.ds(r * C, C)], sem).wait()
            return 0

        lax.fori_loop(c0, c1, drain_c, 0)

    def compute(q, buf_off, carry):
        b4 = q * Q
        _, _, sf, sl = chunk_bounds(q)
        sc_v = plsc.load_gather(bndc_v, [sf])
        ec_v = plsc.load_gather(bndc_v, [sl + 1])
        v0 = jnp.max(sc_v >> 4)
        v1 = jnp.max((ec_v + (L - 1)) >> 4)
        segr = []
        xr = []
        yr = []
        zr = []
        for r in range(Q):
            p = b4 + r
            sgr = plsc.load_gather(segc_v, [p + zeroi])
            segr.append(jnp.where(p < na, sgr, -2))
            xr.append(plsc.load_gather(txc_v, [p + zeroi]))
            yr.append(plsc.load_gather(tyc_v, [p + zeroi]))
            zr.append(plsc.load_gather(tzc_v, [p + zeroi]))

        def v_body(v, cr):
            acc, pms = cr
            jb = v * L
            jdx = alive_v[pl.ds(jb, L)]
            segcv = segc_v[pl.ds(jb, L)]
            txv = txc_v[pl.ds(jb, L)]
            tyv = tyc_v[pl.ds(jb, L)]
            tzv = tzc_v[pl.ds(jb, L)]
            for r in range(Q):
                inp = plsc.load_gather(qbuf, [jdx + (buf_off + r * N)])
                dx = xr[r] - txv
                dy = yr[r] - tyv
                dz = zr[r] - tzv
                sq = dx * dx + dy * dy + dz * dz
                d = _sqrt16(sq)
                pm = jnp.where(segcv == segr[r], 1.0, 0.0)
                e = pm * (inp - d)
                acc = acc + e * e
                pms = pms + pm
            return acc, pms

        return lax.fori_loop(v0, v1, v_body, carry)

    def quad_body(qi, carry):
        acc, pms, ic0, ic1 = carry
        q = qi * NW + wid
        par = qi & 1
        buf_off = par * (Q * N)
        nbuf_off = (1 - par) * (Q * N)

        @pl.when(par == 0)
        def _():
            drain(ic0, ic1, sem0)

        @pl.when(par == 1)
        def _():
            drain(ic0, ic1, sem1)

        def do_issue(_):
            nxq = q + NW

            def i0(_):
                return issue(nxq, nbuf_off, sem1)

            def i1(_):
                return issue(nxq, nbuf_off, sem0)

            return lax.cond(par == 0, i0, i1, 0)

        nc0, nc1 = lax.cond(qi + 1 < myq, do_issue,
                            lambda _: (jnp.int32(0), jnp.int32(0)), 0)

        acc, pms = compute(q, buf_off, (acc, pms))
        return acc, pms, nc0, nc1

    def first_issue(_):
        return issue(wid, 0, sem0)

    ic0, ic1 = lax.cond(myq > 0, first_issue,
                        lambda _: (jnp.int32(0), jnp.int32(0)), 0)
    acc, pms, _, _ = lax.fori_loop(0, myq, quad_body, (zero, zero, ic0, ic1))

    oe_v[...] = acc
    op_v[...] = pms
    pltpu.sync_copy(oe_v, oe_hbm.at[wid])
    pltpu.sync_copy(op_v, op_hbm.at[wid])


def kernel(inputs, target, mask, structure_indices):
    mesh = plsc.VectorSubcoreMesh(core_axis_name="c", subcore_axis_name="s",
                                  num_cores=NC, num_subcores=NS)
    f32 = jnp.float32
    i32 = jnp.int32
    oe, op = pl.kernel(
        _body,
        out_type=(jax.ShapeDtypeStruct((NW, L), f32),
                  jax.ShapeDtypeStruct((NW, L), f32)),
        mesh=mesh,
        compiler_params=pltpu.CompilerParams(needs_layout_passes=False),
        scratch_types=[
            pltpu.VMEM((3 * N,), f32),
            pltpu.VMEM((N,), i32),
            pltpu.VMEM((R,), i32),
            pltpu.VMEM((NPAD,), i32),
            pltpu.VMEM((NPAD,), i32),
            pltpu.VMEM((NPAD,), f32),
            pltpu.VMEM((NPAD,), f32),
            pltpu.VMEM((NPAD,), f32),
            pltpu.VMEM((L,), i32),
            pltpu.VMEM((L,), i32),
            pltpu.VMEM((2 * Q * N,), f32),
            pltpu.VMEM((L,), f32),
            pltpu.VMEM((L,), f32),
            pltpu.SemaphoreType.DMA,
            pltpu.SemaphoreType.DMA,
        ],
    )(inputs, target, mask.reshape(N), structure_indices.astype(i32))

    err2 = jnp.sum(oe)
    pmsum = jnp.sum(op)
    return err2 / (N * N) * pmsum / (N * N)


# speedup vs baseline: 1.0035x; 1.0035x over previous
import jax
import jax.numpy as jnp
from jax import lax
from jax.experimental import pallas as pl
from jax.experimental.pallas import tpu as pltpu
from jax.experimental.pallas import tpu_sc as plsc

N = 3072
R = 1024
NSEG = 8
NC, NS = 2, 16
NW = NC * NS
Q = 8
QSH = 3
C = 512
CSH = 9
L = 16
NPAD = N + 2 * L
BIG = 1 << 30


def _sqrt16(sq):
    sqs = jnp.maximum(sq, 1e-30)
    ii = plsc.bitcast(sqs, jnp.int32)
    ii = jnp.int32(0x5F3759DF) - (ii >> 1)
    y = plsc.bitcast(ii, jnp.float32)
    h = sqs * 0.5
    y = y * (1.5 - h * y * y)
    y = y * (1.5 - h * y * y)
    return sq * y


def _body(inp_hbm, tgt_hbm, msk_hbm, si_hbm,
          oe_hbm, op_hbm,
          tgt_v, msk_v, si_v, alive_v, segc_v, txc_v, tyc_v, tzc_v,
          bnd_v, bndc_v, qbuf, oe_v, op_v, sem0, sem1):
    cid = lax.axis_index("c")
    sid = lax.axis_index("s")
    wid = sid * NC + cid

    pltpu.sync_copy(tgt_hbm, tgt_v)
    pltpu.sync_copy(msk_hbm, msk_v)
    pltpu.sync_copy(si_hbm, si_v)

    iota = lax.iota(jnp.int32, L)
    lane0 = iota == 0
    zero = jnp.zeros((L,), jnp.float32)
    zeroi = jnp.zeros((L,), jnp.int32)

    def prefill_body(v, _=None):
        off = v * L
        alive_v[pl.ds(off, L)] = jnp.full((L,), N - 1, jnp.int32)
        segc_v[pl.ds(off, L)] = jnp.full((L,), -1, jnp.int32)
        txc_v[pl.ds(off, L)] = zero
        tyc_v[pl.ds(off, L)] = zero
        tzc_v[pl.ds(off, L)] = zero

    plsc.parallel_loop(0, NPAD // L, unroll=2)(prefill_body)

    bnd_v[...] = jnp.full((L,), N, jnp.int32)
    seg0 = plsc.load_gather(si_v, [zeroi])
    plsc.store_scatter(bnd_v, [seg0], zeroi, mask=lane0)

    def prep_body(v, base):
        j = v * L + iota
        ridx = j // 3
        rem = j - ridx * 3
        nxtr = jnp.minimum(ridx + jnp.where(rem == 2, 1, 0), R - 1)
        cur = plsc.load_gather(si_v, [ridx])
        nxt = plsc.load_gather(si_v, [nxtr])
        plsc.store_scatter(bnd_v, [nxt], j + 1, mask=nxt != cur)
        m16 = msk_v[pl.ds(v * L, L)]
        am = m16 > 0
        pc = plsc.all_reduce_population_count(am)
        cs = plsc.cumsum(m16)
        cidx = base + cs - m16
        plsc.store_scatter(alive_v, [cidx], j, mask=am)
        plsc.store_scatter(segc_v, [cidx], cur, mask=am)
        j3 = j * 3
        plsc.store_scatter(txc_v, [cidx], plsc.load_gather(tgt_v, [j3]),
                           mask=am)
        plsc.store_scatter(tyc_v, [cidx], plsc.load_gather(tgt_v, [j3 + 1]),
                           mask=am)
        plsc.store_scatter(tzc_v, [cidx], plsc.load_gather(tgt_v, [j3 + 2]),
                           mask=am)
        return base + pc

    base_vec = plsc.parallel_loop(0, N // L, unroll=4, carry=zeroi)(prep_body)
    na = jnp.max(base_vec)

    bnd_v[...] = lax.rev(-plsc.cummax(-lax.rev(bnd_v[...], (0,))), (0,))

    bndc_v[...] = jnp.full((L,), BIG, jnp.int32)
    seg0c = plsc.load_gather(segc_v, [zeroi])
    plsc.store_scatter(bndc_v, [seg0c], zeroi, mask=lane0 & (seg0c >= 0))
    segl = plsc.load_gather(segc_v, [jnp.maximum(na - 1, 0) + zeroi])
    plsc.store_scatter(bndc_v, [segl + 1], na + zeroi,
                       mask=lane0 & (segl >= 0))

    def bndc_body(v, _=None):
        j = v * L + iota
        cur = segc_v[pl.ds(v * L, L)]
        nxt = segc_v[pl.ds(v * L + 1, L)]
        plsc.store_scatter(bndc_v, [nxt], j + 1,
                           mask=(nxt != cur) & (nxt >= 0))

    plsc.parallel_loop(0, (na + L) >> 4, unroll=2)(bndc_body)
    bndc_v[...] = lax.rev(-plsc.cummax(-lax.rev(bndc_v[...], (0,))), (0,))

    nq = (na + (Q - 1)) >> QSH
    myq = jnp.maximum((nq - wid + NW - 1) // NW, 0)

    def rowid(p):
        return jnp.max(plsc.load_gather(alive_v, [p + zeroi]))

    def chunk_bounds(q):
        b4 = q * Q
        sf = plsc.load_gather(segc_v, [b4 + zeroi])
        sl = plsc.load_gather(segc_v,
                              [jnp.minimum(b4 + (Q - 1), na - 1) + zeroi])
        s_v = plsc.load_gather(bnd_v, [sf])
        e_v = plsc.load_gather(bnd_v, [sl + 1])
        return jnp.max(s_v >> CSH), jnp.max((e_v + (C - 1)) >> CSH), sf, sl

    def issue(q, buf_off, sem):
        c0, c1, _, _ = chunk_bounds(q)
        b4 = q * Q
        rids = [rowid(b4 + r) for r in range(Q)]

        def issue_c(c, _):
            for r in range(Q):
                pltpu.async_copy(
                    inp_hbm.at[rids[r], pl.ds(c * C, C)],
                    qbuf.at[pl.ds(buf_off + r * N + c * C, C)], sem)
            return 0

        lax.fori_loop(c0, c1, issue_c, 0)
        return c0, c1

    def drain(c0, c1, sem):
        def drain_c(c, _):
            for r in range(Q):
                pltpu.make_async_copy(inp_hbm.at[0, pl.ds(0, C)],
                                      qbuf.at[pl.ds(r * C, C)], sem).wait()
            return 0

        lax.fori_loop(c0, c1, drain_c, 0)

    def compute(q, buf_off, carry):
        b4 = q * Q
        _, cc1, sf, sl = chunk_bounds(q)
        jlim = cc1 * C - 1
        sc_v = plsc.load_gather(bndc_v, [sf])
        ec_v = plsc.load_gather(bndc_v, [sl + 1])
        v0 = jnp.max(sc_v >> 4)
        v1 = jnp.max((ec_v + (L - 1)) >> 4)
        segr = []
        xr = []
        yr = []
        zr = []
        for r in range(Q):
            p = b4 + r
            sgr = plsc.load_gather(segc_v, [p + zeroi])
            segr.append(jnp.where(p < na, sgr, -2))
            xr.append(plsc.load_gather(txc_v, [p + zeroi]))
            yr.append(plsc.load_gather(tyc_v, [p + zeroi]))
            zr.append(plsc.load_gather(tzc_v, [p + zeroi]))

        def v_body(v, cr):
            acc, pms = cr
            jb = v * L
            jdx = jnp.minimum(alive_v[pl.ds(jb, L)], jlim)
            segcv = segc_v[pl.ds(jb, L)]
            txv = txc_v[pl.ds(jb, L)]
            tyv = tyc_v[pl.ds(jb, L)]
            tzv = tzc_v[pl.ds(jb, L)]
            for r in range(Q):
                inp = plsc.load_gather(qbuf, [jdx + (buf_off + r * N)])
                dx = xr[r] - txv
                dy = yr[r] - tyv
                dz = zr[r] - tzv
                sq = dx * dx + dy * dy + dz * dz
                d = _sqrt16(sq)
                pm = jnp.where(segcv == segr[r], 1.0, 0.0)
                e = pm * (inp - d)
                acc = acc + e * e
                pms = pms + pm
            return acc, pms

        return lax.fori_loop(v0, v1, v_body, carry)

    def quad_body(qi, carry):
        acc, pms, ic0, ic1 = carry
        q = qi * NW + wid
        par = qi & 1
        buf_off = par * (Q * N)
        nbuf_off = (1 - par) * (Q * N)

        @pl.when(par == 0)
        def _():
            drain(ic0, ic1, sem0)

        @pl.when(par == 1)
        def _():
            drain(ic0, ic1, sem1)

        def do_issue(_):
            nxq = q + NW

            def i0(_):
                return issue(nxq, nbuf_off, sem1)

            def i1(_):
                return issue(nxq, nbuf_off, sem0)

            return lax.cond(par == 0, i0, i1, 0)

        nc0, nc1 = lax.cond(qi + 1 < myq, do_issue,
                            lambda _: (jnp.int32(0), jnp.int32(0)), 0)

        acc, pms = compute(q, buf_off, (acc, pms))
        return acc, pms, nc0, nc1

    def first_issue(_):
        return issue(wid, 0, sem0)

    ic0, ic1 = lax.cond(myq > 0, first_issue,
                        lambda _: (jnp.int32(0), jnp.int32(0)), 0)
    acc, pms, _, _ = lax.fori_loop(0, myq, quad_body, (zero, zero, ic0, ic1))

    oe_v[...] = acc
    op_v[...] = pms
    pltpu.sync_copy(oe_v, oe_hbm.at[wid])
    pltpu.sync_copy(op_v, op_hbm.at[wid])


def kernel(inputs, target, mask, structure_indices):
    mesh = plsc.VectorSubcoreMesh(core_axis_name="c", subcore_axis_name="s",
                                  num_cores=NC, num_subcores=NS)
    f32 = jnp.float32
    i32 = jnp.int32
    oe, op = pl.kernel(
        _body,
        out_type=(jax.ShapeDtypeStruct((NW, L), f32),
                  jax.ShapeDtypeStruct((NW, L), f32)),
        mesh=mesh,
        compiler_params=pltpu.CompilerParams(needs_layout_passes=False),
        scratch_types=[
            pltpu.VMEM((3 * N,), f32),
            pltpu.VMEM((N,), i32),
            pltpu.VMEM((R,), i32),
            pltpu.VMEM((NPAD,), i32),
            pltpu.VMEM((NPAD,), i32),
            pltpu.VMEM((NPAD,), f32),
            pltpu.VMEM((NPAD,), f32),
            pltpu.VMEM((NPAD,), f32),
            pltpu.VMEM((L,), i32),
            pltpu.VMEM((L,), i32),
            pltpu.VMEM((2 * Q * N,), f32),
            pltpu.VMEM((L,), f32),
            pltpu.VMEM((L,), f32),
            pltpu.SemaphoreType.DMA,
            pltpu.SemaphoreType.DMA,
        ],
    )(inputs, target, mask.reshape(N), structure_indices.astype(i32))

    err2 = jnp.sum(oe)
    pmsum = jnp.sum(op)
    return err2 / (N * N) * pmsum / (N * N)
